# Initial kernel scaffold; baseline (speedup 1.0000x reference)
import jax
import jax.numpy as jnp
from jax.experimental import pallas as pl


def kernel(logits, labels):
    raise NotImplementedError("write your pallas kernel here")


# speedup vs baseline: 1.0202x; 1.0202x over previous
import jax
import jax.numpy as jnp
from jax.experimental import pallas as pl
from jax.experimental.pallas import tpu as pltpu

N_BINS = 11


def _ece_kernel(n_rows, n_cols, nb, logits_ref, labels_ref, bounds_ref,
                ece_ref, accs_ref, confs_ref, cnt_s, asum_s, csum_s):
    i = pl.program_id(0)

    @pl.when(i == 0)
    def _init():
        cnt_s[...] = jnp.zeros_like(cnt_s)
        asum_s[...] = jnp.zeros_like(asum_s)
        csum_s[...] = jnp.zeros_like(csum_s)

    x = logits_ref[...]
    m = jnp.max(x, axis=1, keepdims=True)
    s = jnp.sum(jnp.exp(x - m), axis=1, keepdims=True)
    conf = 1.0 / s
    col = jax.lax.broadcasted_iota(jnp.int32, x.shape, 1)
    pred = jnp.min(jnp.where(x == m, col, n_cols), axis=1, keepdims=True)
    labels = labels_ref[0]
    acc = (pred == labels).astype(jnp.float32)

    lo = bounds_ref[0:1, 0:N_BINS]
    hi = bounds_ref[0:1, 1:N_BINS + 1]
    mask = ((conf > lo) & (conf <= hi)).astype(jnp.float32)
    cnt_s[...] += jnp.sum(mask, axis=0, keepdims=True)
    asum_s[...] += jnp.sum(mask * acc, axis=0, keepdims=True)
    csum_s[...] += jnp.sum(mask * conf, axis=0, keepdims=True)

    @pl.when(i == nb - 1)
    def _fin():
        cnt = cnt_s[...]
        prop = cnt / jnp.float32(n_rows)
        safe = jnp.maximum(cnt, 1.0)
        acc_in = asum_s[...] / safe
        conf_in = csum_s[...] / safe
        nonempty = cnt > 0
        contrib = jnp.where(nonempty, jnp.abs(conf_in - acc_in) * prop, 0.0)
        ece_ref[...] = jnp.sum(contrib, axis=1, keepdims=True)
        accs_ref[...] = jnp.where(nonempty, acc_in, 0.0)
        confs_ref[...] = jnp.where(nonempty, conf_in, 0.0)


def kernel(logits, labels):
    n_rows, n_cols = logits.shape
    block_r = 512
    nb = n_rows // block_r
    labels3 = labels.reshape(nb, block_r, 1)
    bounds = jnp.linspace(0.0, 1.0, N_BINS + 1).astype(jnp.float32)
    bounds = bounds.reshape(1, N_BINS + 1)

    import functools
    body = functools.partial(_ece_kernel, n_rows, n_cols, nb)
    ece2, accs2, confs2 = pl.pallas_call(
        body,
        grid=(nb,),
        in_specs=[
            pl.BlockSpec((block_r, n_cols), lambda i: (i, 0)),
            pl.BlockSpec((1, block_r, 1), lambda i: (i, 0, 0)),
            pl.BlockSpec((1, N_BINS + 1), lambda i: (0, 0)),
        ],
        out_specs=[
            pl.BlockSpec((1, 1), lambda i: (0, 0)),
            pl.BlockSpec((1, N_BINS), lambda i: (0, 0)),
            pl.BlockSpec((1, N_BINS), lambda i: (0, 0)),
        ],
        out_shape=[
            jax.ShapeDtypeStruct((1, 1), jnp.float32),
            jax.ShapeDtypeStruct((1, N_BINS), jnp.float32),
            jax.ShapeDtypeStruct((1, N_BINS), jnp.float32),
        ],
        scratch_shapes=[
            pltpu.VMEM((1, N_BINS), jnp.float32),
            pltpu.VMEM((1, N_BINS), jnp.float32),
            pltpu.VMEM((1, N_BINS), jnp.float32),
        ],
    )(logits, labels3, bounds)
    return (ece2.reshape(1), accs2.reshape(N_BINS), confs2.reshape(N_BINS))
